# Initial kernel scaffold; baseline (speedup 1.0000x reference)
import functools

import jax
import jax.numpy as jnp
from jax import lax
from jax.experimental import pallas as pl
from jax.experimental.pallas import tpu as pltpu
from jax.experimental.pallas import tpu_sc as plsc

_N = 50000
_D = 256
_C = 1024

_NC = 2
_NS = 16
_NW = _NC * _NS
_CPW = 13
_Q = _CPW * 128
_NP = _NW * _Q
_A = 1040
_RPT = _A // _NS

_BN = 416


def _proj_body(x_ref, wg_ref, bg_ref, gg_ref, gb_ref,
               wf_ref, bf_ref, fg_ref, fb_ref, o_ref):
    x = x_ref[...]

    def ln(h, gamma, beta):
        mu = jnp.mean(h, axis=1, keepdims=True)
        var = jnp.mean((h - mu) ** 2, axis=1, keepdims=True)
        return (h - mu) * lax.rsqrt(var + 1e-5) * gamma + beta

    hg = jnp.dot(x, wg_ref[...], preferred_element_type=jnp.float32) + bg_ref[...]
    gates = jax.nn.sigmoid(ln(hg, gg_ref[...], gb_ref[...]))

    hf = jnp.dot(x, wf_ref[...], preferred_element_type=jnp.float32) + bf_ref[...]
    hf = ln(hf, fg_ref[...], fb_ref[...])
    feats = 0.5 * hf * (1.0 + lax.erf(hf * 0.7071067811865476))

    o_ref[...] = gates * feats


def _proj(x_p, wgt, bg, gg, gb, wft, bf, fg, fb):
    row_spec = pl.BlockSpec((_BN, _D), lambda i: (i, 0))
    mat_spec = pl.BlockSpec((_D, _D), lambda i: (0, 0))
    vec_spec = pl.BlockSpec((1, _D), lambda i: (0, 0))
    return pl.pallas_call(
        _proj_body,
        grid=(_NP // _BN,),
        in_specs=[row_spec, mat_spec, vec_spec, vec_spec, vec_spec,
                  mat_spec, vec_spec, vec_spec, vec_spec],
        out_specs=row_spec,
        out_shape=jax.ShapeDtypeStruct((_NP, _D), jnp.float32),
    )(x_p, wgt, bg, gg, gb, wft, bf, fg, fb)


_MESH = plsc.VectorSubcoreMesh(core_axis_name="c", subcore_axis_name="s",
                               num_cores=_NC, num_subcores=_NS)


@functools.partial(
    pl.kernel,
    out_type=(jax.ShapeDtypeStruct((_NC, _A, _D), jnp.float32),
              jax.ShapeDtypeStruct((_NC, _A, 16), jnp.float32)),
    mesh=_MESH,
    scratch_types=[
        pltpu.VMEM((_CPW, 128), jnp.int32),
        pltpu.VMEM((128, _D), jnp.float32),
        pltpu.VMEM((128, 16), jnp.float32),
        pltpu.VMEM_SHARED((_A, _D), jnp.float32),
        pltpu.VMEM_SHARED((_A, 16), jnp.float32),
    ],
)
def _sc_scatter(gated_hbm, ca2_hbm, zsum_hbm, zcnt_hbm, ones_hbm,
                sums_out, cnt_out, idx_v, rows_v, ones_v, ssum, scnt):
    c = lax.axis_index("c")
    s = lax.axis_index("s")
    w = s * _NC + c

    pltpu.sync_copy(zsum_hbm.at[pl.ds(s * _RPT, _RPT)], ssum.at[pl.ds(s * _RPT, _RPT)])
    pltpu.sync_copy(zcnt_hbm.at[pl.ds(s * _RPT, _RPT)], scnt.at[pl.ds(s * _RPT, _RPT)])
    pltpu.sync_copy(ones_hbm, ones_v)
    plsc.subcore_barrier()

    pltpu.sync_copy(ca2_hbm.at[pl.ds(w * _CPW, _CPW)], idx_v)
    base = w * _Q

    def body(j, carry):
        pltpu.sync_copy(gated_hbm.at[pl.ds(base + j * 128, 128)], rows_v)
        pltpu.sync_copy(rows_v, ssum.at[idx_v.at[j]], add=True)
        pltpu.sync_copy(ones_v, scnt.at[idx_v.at[j]], add=True)
        return carry

    lax.fori_loop(0, _CPW, body, 0)
    plsc.subcore_barrier()

    pltpu.sync_copy(ssum.at[pl.ds(s * _RPT, _RPT)],
                    sums_out.at[c].at[pl.ds(s * _RPT, _RPT)])
    pltpu.sync_copy(scnt.at[pl.ds(s * _RPT, _RPT)],
                    cnt_out.at[c].at[pl.ds(s * _RPT, _RPT)])


def _combine_body(s_ref, c_ref, o_ref):
    sums = s_ref[0] + s_ref[1]
    cnt = c_ref[0, :, 0] + c_ref[1, :, 0]
    o_ref[...] = sums / jnp.maximum(cnt, 1.0)[:, None]


def _combine(sums, cnts):
    return pl.pallas_call(
        _combine_body,
        out_shape=jax.ShapeDtypeStruct((_A, _D), jnp.float32),
    )(sums, cnts)


@functools.partial(
    pl.kernel,
    out_type=jax.ShapeDtypeStruct((_NP, _D), jnp.float32),
    mesh=_MESH,
    scratch_types=[
        pltpu.VMEM((_CPW, 128), jnp.int32),
        pltpu.VMEM((128, _D), jnp.float32),
    ],
)
def _sc_gather(pooled_hbm, ca2_hbm, out_hbm, idx_v, rows_v):
    c = lax.axis_index("c")
    s = lax.axis_index("s")
    w = s * _NC + c

    pltpu.sync_copy(ca2_hbm.at[pl.ds(w * _CPW, _CPW)], idx_v)
    base = w * _Q

    def body(j, carry):
        pltpu.sync_copy(pooled_hbm.at[idx_v.at[j]], rows_v)
        pltpu.sync_copy(rows_v, out_hbm.at[pl.ds(base + j * 128, 128)])
        return carry

    lax.fori_loop(0, _CPW, body, 0)


def kernel(x, cluster_assignments, batch, Wg, bg, g_gamma, g_beta,
           Wf, bf, f_gamma, f_beta):
    del batch

    x_p = jnp.zeros((_NP, _D), jnp.float32).at[:_N].set(x)
    ca_p = jnp.full((_NP,), _C, jnp.int32).at[:_N].set(cluster_assignments)
    ca2 = ca_p.reshape(_NP // 128, 128)

    gated = _proj(x_p, Wg.T, bg.reshape(1, _D), g_gamma.reshape(1, _D),
                  g_beta.reshape(1, _D), Wf.T, bf.reshape(1, _D),
                  f_gamma.reshape(1, _D), f_beta.reshape(1, _D))

    zsum = jnp.zeros((_A, _D), jnp.float32)
    zcnt = jnp.zeros((_A, 16), jnp.float32)
    ones = jnp.ones((128, 16), jnp.float32)
    sums, cnts = _sc_scatter(gated, ca2, zsum, zcnt, ones)

    pooled = _combine(sums, cnts)

    out_p = _sc_gather(pooled, ca2)
    return out_p[:_N]


# speedup vs baseline: 1.8284x; 1.8284x over previous
import functools

import jax
import jax.numpy as jnp
from jax import lax
from jax.experimental import pallas as pl
from jax.experimental.pallas import tpu as pltpu
from jax.experimental.pallas import tpu_sc as plsc

_N = 50000
_D = 256
_C = 1024

_NC = 2
_NS = 16
_NW = _NC * _NS
_CPW = 13
_Q = _CPW * 128
_NP = _NW * _Q
_A = 1152

_BN = 416
_NB = _NP // _BN


def _proj_pool_body(ids_ref, x_ref, wg_ref, bg_ref, gg_ref, gb_ref,
                    wf_ref, bf_ref, fg_ref, fb_ref, o_ref,
                    acc_ref, cnt_ref):
    i = pl.program_id(0)
    x = x_ref[...]

    def ln(h, gamma, beta):
        mu = jnp.mean(h, axis=1, keepdims=True)
        var = jnp.mean((h - mu) ** 2, axis=1, keepdims=True)
        return (h - mu) * lax.rsqrt(var + 1e-5) * gamma + beta

    hg = jnp.dot(x, wg_ref[...], preferred_element_type=jnp.float32) + bg_ref[...]
    gates = jax.nn.sigmoid(ln(hg, gg_ref[...], gb_ref[...]))

    hf = jnp.dot(x, wf_ref[...], preferred_element_type=jnp.float32) + bf_ref[...]
    hf = ln(hf, fg_ref[...], fb_ref[...])
    feats = 0.5 * hf * (1.0 + lax.erf(hf * 0.7071067811865476))

    gated = gates * feats

    ids = ids_ref[0]
    clusters = lax.broadcasted_iota(jnp.int32, (_A, _BN), 0)
    oh_t = (clusters == ids).astype(jnp.bfloat16)
    sums_part = jax.lax.dot_general(
        oh_t, gated.astype(jnp.bfloat16),
        dimension_numbers=(((1,), (0,)), ((), ())),
        preferred_element_type=jnp.float32)
    cnt_part = jax.lax.dot_general(
        oh_t, jnp.ones((_BN, 8), jnp.bfloat16),
        dimension_numbers=(((1,), (0,)), ((), ())),
        preferred_element_type=jnp.float32)

    @pl.when(i == 0)
    def _init():
        acc_ref[...] = jnp.zeros_like(acc_ref)
        cnt_ref[...] = jnp.zeros_like(cnt_ref)

    acc_ref[...] += sums_part
    cnt_ref[...] += cnt_part

    @pl.when(i == _NB - 1)
    def _finish():
        cnt = jnp.maximum(cnt_ref[:, 0], 1.0)
        o_ref[...] = acc_ref[...] / cnt[:, None]


def _proj_pool(ids3, x_p, wgt, bg, gg, gb, wft, bf, fg, fb):
    row_spec = pl.BlockSpec((_BN, _D), lambda i: (i, 0))
    mat_spec = pl.BlockSpec((_D, _D), lambda i: (0, 0))
    vec_spec = pl.BlockSpec((1, _D), lambda i: (0, 0))
    ids_spec = pl.BlockSpec((1, 1, _BN), lambda i: (i, 0, 0))
    return pl.pallas_call(
        _proj_pool_body,
        grid=(_NB,),
        in_specs=[ids_spec, row_spec, mat_spec, vec_spec, vec_spec, vec_spec,
                  mat_spec, vec_spec, vec_spec, vec_spec],
        out_specs=pl.BlockSpec((_A, _D), lambda i: (0, 0)),
        out_shape=jax.ShapeDtypeStruct((_A, _D), jnp.float32),
        scratch_shapes=[
            pltpu.VMEM((_A, _D), jnp.float32),
            pltpu.VMEM((_A, 8), jnp.float32),
        ],
    )(ids3, x_p, wgt, bg, gg, gb, wft, bf, fg, fb)


_MESH = plsc.VectorSubcoreMesh(core_axis_name="c", subcore_axis_name="s",
                               num_cores=_NC, num_subcores=_NS)


@functools.partial(
    pl.kernel,
    out_type=jax.ShapeDtypeStruct((_NP, _D), jnp.float32),
    mesh=_MESH,
    scratch_types=[
        pltpu.VMEM((_CPW, 128), jnp.int32),
        pltpu.VMEM((128, _D), jnp.float32),
    ],
)
def _sc_gather(pooled_hbm, ca3_hbm, out_hbm, idx_v, rows_v):
    c = lax.axis_index("c")
    s = lax.axis_index("s")
    w = s * _NC + c

    pltpu.sync_copy(ca3_hbm.at[w], idx_v)
    base = w * _Q

    def body(j, carry):
        pltpu.sync_copy(pooled_hbm.at[idx_v.at[j]], rows_v)
        pltpu.sync_copy(rows_v, out_hbm.at[pl.ds(base + j * 128, 128)])
        return carry

    lax.fori_loop(0, _CPW, body, 0)


def kernel(x, cluster_assignments, batch, Wg, bg, g_gamma, g_beta,
           Wf, bf, f_gamma, f_beta):
    del batch

    x_p = jnp.zeros((_NP, _D), jnp.float32).at[:_N].set(x)
    ca_p = jnp.full((_NP,), _C, jnp.int32).at[:_N].set(cluster_assignments)
    ids3 = ca_p.reshape(_NB, 1, _BN)
    ca3 = ca_p.reshape(_NW, _CPW, 128)

    pooled = _proj_pool(ids3, x_p, Wg.T, bg.reshape(1, _D),
                        g_gamma.reshape(1, _D), g_beta.reshape(1, _D),
                        Wf.T, bf.reshape(1, _D), f_gamma.reshape(1, _D),
                        f_beta.reshape(1, _D))

    out_p = _sc_gather(pooled, ca3)
    return out_p[:_N]
